# baseline (speedup 1.0000x reference)
import functools

import jax
import jax.numpy as jnp
from jax import lax
from jax.experimental import pallas as pl
from jax.experimental.pallas import tpu as pltpu
from jax.experimental.pallas import tpu_sc as plsc

_B = 16384
_U = 64
_D = 128
_NC = 2
_NS = 16
_NW = _NC * _NS
_BPW = _B // _NW
_L = 16
_UB = _U // _L
_CH = 256
_NG = _CH // _L
_NBM = _D // 32

_I32 = jnp.int32
_F32 = jnp.float32

_GATHER_DNUMS = lax.GatherDimensionNumbers(
    offset_dims=(), collapsed_slice_dims=(0,), start_index_map=(0,)
)


def _shuffle(v, idx):
    return lax.gather(
        v, idx[:, None], _GATHER_DNUMS, (1,),
        mode=lax.GatherScatterMode.PROMISE_IN_BOUNDS,
    )


def _max_all(v, lanes):
    for s in (1, 2, 4, 8):
        v = jnp.maximum(v, _shuffle(v, lanes ^ s))
    return v[0]


def _popcount32(v):
    shr = lax.shift_right_logical
    v = v - (shr(v, 1) & _I32(0x55555555))
    v = (v & _I32(0x33333333)) + (shr(v, 2) & _I32(0x33333333))
    v = (v + shr(v, 4)) & _I32(0x0F0F0F0F)
    return shr(v * _I32(0x01010101), 24)


@functools.partial(
    pl.kernel,
    out_type=jax.ShapeDtypeStruct((_B, _U), jnp.float32),
    mesh=plsc.VectorSubcoreMesh(core_axis_name="c", subcore_axis_name="s"),
    scratch_types=[
        pltpu.VMEM((_D, _CH), jnp.float32),
        pltpu.VMEM((_D, _U), jnp.float32),
        pltpu.VMEM((_CH, _U), jnp.float32),
    ],
)
def _trop_top2_sc(xt_hbm, wt_hbm, out_hbm, x_t_v, wt_v, o_v):
    wid = lax.axis_index("s") * _NC + lax.axis_index("c")
    base = wid * _BPW
    pltpu.sync_copy(wt_hbm, wt_v)

    neg = jnp.full((_L,), -jnp.inf, _F32)
    pos = jnp.full((_L,), jnp.inf, _F32)
    zero_i = jnp.zeros((_L,), _I32)
    lanes = lax.broadcasted_iota(_I32, (_L,), 0)

    def wscan(jj, carry):
        wmx, wmn = carry
        for ub in range(_UB):
            vv = wt_v[jj, pl.ds(ub * _L, _L)]
            wmx = jnp.maximum(wmx, vv)
            wmn = jnp.minimum(wmn, vv)
        return wmx, wmn

    wmx, wmn = lax.fori_loop(0, _D, wscan, (neg, pos))
    w_spread = _max_all(wmx, lanes) + _max_all(-wmn, lanes)
    w_spread_vec = jnp.full((_L,), 0.0, _F32) + w_spread

    def rowgroup(rg, carry):
        cols = pl.ds(rg * _L, _L)
        def topx(jj, ms):
            m1, m2 = ms
            v = x_t_v[jj, cols]
            lo = jnp.minimum(m1, v)
            return jnp.maximum(m1, v), jnp.maximum(m2, lo)

        xm1, xm2 = lax.fori_loop(0, _D, topx, (neg, neg))
        thresh = xm2 - w_spread_vec

        bmv = []
        for g in range(_NBM):
            def bmstep(kk, bm, g=g):
                for k8 in range(8):
                    one = jnp.full((_L,), _I32(1))
                    sh = jnp.full((_L,), kk * 8 + k8, _I32)
                    jj = g * 32 + kk * 8 + k8
                    bit = lax.shift_left(one, sh)
                    bm = bm | jnp.where(x_t_v[jj, cols] >= thresh, bit, zero_i)
                return bm

            bmv.append(lax.fori_loop(0, 4, bmstep, zero_i))

        def rowpair(r2, rcarry):
            ra = r2 * 2
            rb = ra + 1
            rota = (lanes + ra) & (_L - 1)
            rotb = (lanes + rb) & (_L - 1)
            spa = jnp.full((_L,), ra, _I32)
            spb = jnp.full((_L,), rb, _I32)
            bma = [_shuffle(bmv[g], rota)[0] for g in range(_NBM)]
            bmb = [_shuffle(bmv[g], rotb)[0] for g in range(_NBM)]
            ms = [neg] * (4 * _UB)

            for g in range(_NBM):
                trip = jnp.maximum(_popcount32(bma[g]), _popcount32(bmb[g]))

                def gbody(i, c, g=g):
                    wa, wb = c[0], c[1]
                    cms = list(c[2:])
                    words = []
                    for half, (w, sp) in enumerate(((wa, spa), (wb, spb))):
                        low = w & (-w)
                        found = w != _I32(0)
                        bits = lax.bitcast_convert_type(low.astype(_F32), _I32)
                        jloc = lax.shift_right_logical(bits, 23) & 255
                        j = jnp.where(found, jloc + _I32(g * 32 - 127), _I32(0))
                        words.append(w & (w - _I32(1)))
                        xrow = x_t_v[j, cols]
                        xs = jnp.where(found, _shuffle(xrow, sp), neg)
                        o = half * 2 * _UB
                        for ub in range(_UB):
                            v = wt_v[j, pl.ds(ub * _L, _L)] + xs
                            lo = jnp.minimum(cms[o + ub], v)
                            cms[o + ub] = jnp.maximum(cms[o + ub], v)
                            cms[o + _UB + ub] = jnp.maximum(cms[o + _UB + ub], lo)
                    return (words[0], words[1]) + tuple(cms)

                res = lax.fori_loop(0, trip, gbody, (bma[g], bmb[g]) + tuple(ms))
                ms = list(res[2:])

            for ub in range(_UB):
                o_v[rg * _L + ra, pl.ds(ub * _L, _L)] = ms[ub] - ms[_UB + ub]
                o_v[rg * _L + rb, pl.ds(ub * _L, _L)] = (
                    ms[2 * _UB + ub] - ms[3 * _UB + ub]
                )
            return rcarry

        lax.fori_loop(0, _L // 2, rowpair, 0)
        return carry

    for chunk in range(_BPW // _CH):
        cbase = base + chunk * _CH
        pltpu.sync_copy(xt_hbm.at[:, pl.ds(cbase, _CH)], x_t_v)
        lax.fori_loop(0, _NG, rowgroup, 0)
        pltpu.sync_copy(o_v, out_hbm.at[pl.ds(cbase, _CH)])


def kernel(inputs, w):
    return _trop_top2_sc(inputs.T, w.T)


# speedup vs baseline: 1.2393x; 1.2393x over previous
import functools

import jax
import jax.numpy as jnp
from jax import lax
from jax.experimental import pallas as pl
from jax.experimental.pallas import tpu as pltpu
from jax.experimental.pallas import tpu_sc as plsc

_B = 16384
_U = 64
_D = 128
_NC = 2
_NS = 16
_NW = _NC * _NS
_BPW = _B // _NW
_L = 16
_UB = _U // _L
_CH = 256
_NG = _CH // _L
_T = 8
_NBM = _D // 32

_I32 = jnp.int32
_F32 = jnp.float32

_GATHER_DNUMS = lax.GatherDimensionNumbers(
    offset_dims=(), collapsed_slice_dims=(0,), start_index_map=(0,)
)


def _shuffle(v, idx):
    return lax.gather(
        v, idx[:, None], _GATHER_DNUMS, (1,),
        mode=lax.GatherScatterMode.PROMISE_IN_BOUNDS,
    )


def _max_all(v, lanes):
    for s in (1, 2, 4, 8):
        v = jnp.maximum(v, _shuffle(v, lanes ^ s))
    return v[0]


def _popcount32(v):
    shr = lax.shift_right_logical
    v = v - (shr(v, 1) & _I32(0x55555555))
    v = (v & _I32(0x33333333)) + (shr(v, 2) & _I32(0x33333333))
    v = (v + shr(v, 4)) & _I32(0x0F0F0F0F)
    return shr(v * _I32(0x01010101), 24)


def _pop_candidate(bms, rg, r_splat, x_t_v, neg):
    nz = [b != 0 for b in bms]
    sel = jnp.where(
        nz[0], bms[0], jnp.where(nz[1], bms[1], jnp.where(nz[2], bms[2], bms[3]))
    )
    base = jnp.where(
        nz[0],
        _I32(0),
        jnp.where(nz[1], _I32(32), jnp.where(nz[2], _I32(64), _I32(96))),
    )
    low = sel & (-sel)
    bits = lax.bitcast_convert_type(low.astype(_F32), _I32)
    jloc = ((bits >> 23) & 255) - 127
    found = sel != _I32(0)
    j = jnp.where(found, base + jloc, _I32(0))
    cleared = sel & (sel - 1)
    g0 = nz[0]
    g1 = jnp.logical_and(jnp.logical_not(nz[0]), nz[1])
    g2 = jnp.logical_and(jnp.logical_not(jnp.logical_or(nz[0], nz[1])), nz[2])
    g3 = jnp.logical_not(jnp.logical_or(jnp.logical_or(nz[0], nz[1]), nz[2]))
    new_bms = (
        jnp.where(g0, cleared, bms[0]),
        jnp.where(g1, cleared, bms[1]),
        jnp.where(g2, cleared, bms[2]),
        jnp.where(g3, cleared, bms[3]),
    )
    xrow = x_t_v[j, pl.ds(rg * _L, _L)]
    xs_vec = jnp.where(found, _shuffle(xrow, r_splat), neg)
    return new_bms, j, xs_vec


@functools.partial(
    pl.kernel,
    out_type=jax.ShapeDtypeStruct((_B, _U), jnp.float32),
    mesh=plsc.VectorSubcoreMesh(core_axis_name="c", subcore_axis_name="s"),
    scratch_types=[
        pltpu.VMEM((_D, _CH), jnp.float32),
        pltpu.VMEM((_D, _U), jnp.float32),
        pltpu.VMEM((_CH, _U), jnp.float32),
    ],
)
def _trop_top2_sc(xt_hbm, wt_hbm, out_hbm, x_t_v, wt_v, o_v):
    wid = lax.axis_index("s") * _NC + lax.axis_index("c")
    base = wid * _BPW
    pltpu.sync_copy(wt_hbm, wt_v)

    neg = jnp.full((_L,), -jnp.inf, _F32)
    pos = jnp.full((_L,), jnp.inf, _F32)
    zero_i = jnp.zeros((_L,), _I32)
    lanes = lax.broadcasted_iota(_I32, (_L,), 0)

    def wscan(jj, carry):
        wmx, wmn = carry
        for ub in range(_UB):
            vv = wt_v[jj, pl.ds(ub * _L, _L)]
            wmx = jnp.maximum(wmx, vv)
            wmn = jnp.minimum(wmn, vv)
        return wmx, wmn

    wmx, wmn = lax.fori_loop(0, _D, wscan, (neg, pos))
    w_spread = _max_all(wmx, lanes) + _max_all(-wmn, lanes)
    w_spread_vec = jnp.full((_L,), 0.0, _F32) + w_spread

    def rowgroup(rg, carry):
        cols = pl.ds(rg * _L, _L)
        def topx(jj, ms):
            m1, m2 = ms
            v = x_t_v[jj, cols]
            lo = jnp.minimum(m1, v)
            return jnp.maximum(m1, v), jnp.maximum(m2, lo)

        xm1, xm2 = lax.fori_loop(0, _D, topx, (neg, neg))
        thresh = xm2 - w_spread_vec

        bmv = []
        for g in range(_NBM):
            def bmstep(kk, bm, g=g):
                for k8 in range(8):
                    one = jnp.full((_L,), _I32(1))
                    sh = jnp.full((_L,), kk * 8 + k8, _I32)
                    jj = g * 32 + kk * 8 + k8
                    bit = lax.shift_left(one, sh)
                    bm = bm | jnp.where(x_t_v[jj, cols] >= thresh, bit, zero_i)
                return bm

            bmv.append(lax.fori_loop(0, 4, bmstep, zero_i))

        _NR = 4

        def rowquad(q, rcarry):
            r0 = q * _NR
            rows = [r0 + k for k in range(_NR)]
            sps = [jnp.full((_L,), r, _I32) for r in rows]
            bms = []
            trips = []
            for r in rows:
                rot = (lanes + r) & (_L - 1)
                wrds = [_shuffle(bmv[g], rot)[0] for g in range(_NBM)]
                bms.extend(wrds)
                n = _popcount32(wrds[0])
                for g in range(1, _NBM):
                    n = n + _popcount32(wrds[g])
                trips.append(n)
            trip = trips[0]
            for k in range(1, _NR):
                trip = jnp.maximum(trip, trips[k])

            def body(i, c):
                cbms = list(c[: _NR * _NBM])
                cms = list(c[_NR * _NBM:])
                for k in range(_NR):
                    wr = cbms[k * _NBM:(k + 1) * _NBM]
                    wr, j2, xs2 = _pop_candidate(wr, rg, sps[k], x_t_v, neg)
                    cbms[k * _NBM:(k + 1) * _NBM] = list(wr)
                    o = k * 2 * _UB
                    for ub in range(_UB):
                        v = wt_v[j2, pl.ds(ub * _L, _L)] + xs2
                        lo = jnp.minimum(cms[o + ub], v)
                        cms[o + ub] = jnp.maximum(cms[o + ub], v)
                        cms[o + _UB + ub] = jnp.maximum(cms[o + _UB + ub], lo)
                return tuple(cbms) + tuple(cms)

            init = tuple(bms) + tuple([neg] * (_NR * 2 * _UB))
            res = lax.fori_loop(0, trip, body, init)
            cms = res[_NR * _NBM:]
            for k in range(_NR):
                o = k * 2 * _UB
                for ub in range(_UB):
                    o_v[rg * _L + rows[k], pl.ds(ub * _L, _L)] = (
                        cms[o + ub] - cms[o + _UB + ub]
                    )
            return rcarry

        lax.fori_loop(0, _L // _NR, rowquad, 0)
        return carry

    for chunk in range(_BPW // _CH):
        cbase = base + chunk * _CH
        pltpu.sync_copy(xt_hbm.at[:, pl.ds(cbase, _CH)], x_t_v)
        lax.fori_loop(0, _NG, rowgroup, 0)
        pltpu.sync_copy(o_v, out_hbm.at[pl.ds(cbase, _CH)])


def kernel(inputs, w):
    return _trop_top2_sc(inputs.T, w.T)
